# baseline (speedup 1.0000x reference)
import functools

import jax
import jax.numpy as jnp
from jax import lax
from jax.experimental import pallas as pl
from jax.experimental.pallas import tpu as pltpu
from jax.experimental.pallas import tpu_sc as plsc

_B, _H, _W, _K, _P, _C = 2, 384, 384, 8, 100000, 32
_N = _B * _H * _W
_NW = 32
_NROW = _B * _H
_RPW = _NROW // _NW
_RW = _K * _W
_CW = _C // 2
_OROW = _C * _W
_NG = _W // 16


def _body(dists_hbm, idx_hbm, radii_hbm, feat_hbm, out_hbm,
          radii_v, ibuf, wbuf, rows0, rows1, obuf, semA, semB0, semB1, semD):
    wid = lax.axis_index("c") * 16 + lax.axis_index("s")
    pltpu.sync_copy(radii_hbm, radii_v)

    lane = lax.iota(jnp.int32, 16)
    zvec = lane * 0

    def issue_a_idx(r):
        base = (wid * _RPW + r) * _RW
        pltpu.async_copy(idx_hbm.at[pl.ds(base, _RW)], ibuf, semA)

    def issue_a_dists(r):
        base = (wid * _RPW + r) * _RW
        pltpu.async_copy(dists_hbm.at[pl.ds(base, _RW)], wbuf, semA)

    def wait_a():
        pltpu.make_async_copy(idx_hbm.at[pl.ds(0, _RW)], ibuf, semA).wait()
        pltpu.make_async_copy(dists_hbm.at[pl.ds(0, _RW)], wbuf, semA).wait()

    def issue_b(k, rows, semB):
        for j in range(_W // 128):
            pltpu.async_copy(
                feat_hbm.at[ibuf.at[pl.ds(k * _W + j * 128, 128)]],
                rows.at[pl.ds(j * 128, 128)], semB)

    def wait_b(rows, semB):
        for j in range(_W // 128):
            pltpu.make_async_copy(
                feat_hbm.at[ibuf.at[pl.ds(0, 128)]],
                rows.at[pl.ds(j * 128, 128)], semB).wait()

    def issue_d(r):
        base = (wid * _RPW + r) * _OROW
        pltpu.async_copy(obuf, out_hbm.at[pl.ds(base, _OROW)], semD)

    def wait_d():
        pltpu.make_async_copy(obuf, out_hbm.at[pl.ds(0, _OROW)], semD).wait()

    def weights():
        @plsc.parallel_loop(0, _NG, unroll=1)
        def wgroup(g):
            o = g * 16
            ws = []
            for k in range(_K):
                iv = ibuf[pl.ds(k * _W + o, 16)]
                r = plsc.load_gather(radii_v, [iv])
                ws.append(1.0 - wbuf[pl.ds(k * _W + o, 16)] / (r * r))
            pairs = ws
            while len(pairs) > 1:
                pairs = [a + b for a, b in zip(pairs[0::2], pairs[1::2])]
            rcp = 1.0 / jnp.maximum(pairs[0], 1e-10)
            for k in range(_K):
                wbuf[pl.ds(k * _W + o, 16)] = ws[k] * rcp

    def accum(k, rows):
        first = k == 0

        @plsc.parallel_loop(0, _NG, unroll=1)
        def _wg(g):
            o = g * 16
            wk = wbuf[pl.ds(k * _W + o, 16)]
            rowv = lane + o
            for ch in range(_CW):
                col = plsc.load_gather(rows, [rowv, zvec + ch])
                ev, od = plsc.unpack(plsc.bitcast(col, jnp.bfloat16),
                                     format=plsc.PackFormat.INTERLEAVED)
                pe, po = wk * ev, wk * od
                se = obuf.at[pl.ds(2 * ch * _W + o, 16)]
                so = obuf.at[pl.ds((2 * ch + 1) * _W + o, 16)]
                if first:
                    se[...] = pe
                    so[...] = po
                else:
                    plsc.addupdate(se, pe)
                    plsc.addupdate(so, po)

    issue_a_idx(0)
    issue_a_dists(0)

    def row(r, carry):
        wait_a()
        issue_b(0, rows0, semB0)
        issue_b(1, rows1, semB1)
        weights()

        @pl.when(r > 0)
        def _():
            wait_d()

        for k in range(_K):
            rows, semB = (rows0, semB0) if k % 2 == 0 else (rows1, semB1)
            wait_b(rows, semB)
            accum(k, rows)
            if k + 2 < _K:
                issue_b(k + 2, rows, semB)
            if k == _K - 3:
                @pl.when(r < _RPW - 1)
                def _():
                    issue_a_idx(r + 1)

        @pl.when(r < _RPW - 1)
        def _():
            issue_a_dists(r + 1)

        issue_d(r)
        return carry

    lax.fori_loop(0, _RPW, row, 0)
    wait_d()


@jax.jit
def _render(dt, it, radii, feat_packed):
    mesh = plsc.VectorSubcoreMesh(core_axis_name="c", subcore_axis_name="s")
    f = pl.kernel(
        _body,
        out_type=jax.ShapeDtypeStruct((_N * _C,), jnp.float32),
        mesh=mesh,
        scratch_types=[
            pltpu.VMEM((_P,), jnp.float32),
            pltpu.VMEM((_RW,), jnp.int32),
            pltpu.VMEM((_RW,), jnp.float32),
            pltpu.VMEM((_W, _CW), jnp.int32),
            pltpu.VMEM((_W, _CW), jnp.int32),
            pltpu.VMEM((_OROW,), jnp.float32),
            pltpu.SemaphoreType.DMA,
            pltpu.SemaphoreType.DMA,
            pltpu.SemaphoreType.DMA,
            pltpu.SemaphoreType.DMA,
        ],
        compiler_params=pltpu.CompilerParams(
            needs_layout_passes=False, use_tc_tiling_on_sc=False),
    )
    return f(dt, it, radii, feat_packed)


def kernel(dists, idx, radii, features):
    dt = dists.transpose(0, 1, 3, 2).reshape(-1)
    it = idx.transpose(0, 1, 3, 2).reshape(-1)
    feat_packed = lax.bitcast_convert_type(
        features.astype(jnp.bfloat16).reshape(_P, _CW, 2), jnp.int32)
    out = _render(dt, it, radii, feat_packed)
    return out.reshape(_B, _H, _C, _W).transpose(0, 1, 3, 2)


# speedup vs baseline: 1.0024x; 1.0024x over previous
import functools

import jax
import jax.numpy as jnp
from jax import lax
from jax.experimental import pallas as pl
from jax.experimental.pallas import tpu as pltpu
from jax.experimental.pallas import tpu_sc as plsc

_B, _H, _W, _K, _P, _C = 2, 384, 384, 8, 100000, 32
_N = _B * _H * _W
_NW = 32
_NROW = _B * _H
_RPW = _NROW // _NW
_RW = _K * _W
_CW = _C // 2
_OROW = _C * _W
_NG = _W // 16


def _body(dists_hbm, idx_hbm, radii_hbm, feat_hbm, out_hbm,
          radii_v, ibuf, wbuf, rows0, rows1, obuf, semA, semB0, semB1, semD):
    wid = lax.axis_index("c") * 16 + lax.axis_index("s")
    pltpu.sync_copy(radii_hbm, radii_v)

    lane = lax.iota(jnp.int32, 16)
    zvec = lane * 0

    def issue_a_idx(r):
        base = (wid * _RPW + r) * _RW
        pltpu.async_copy(idx_hbm.at[pl.ds(base, _RW)], ibuf, semA)

    def issue_a_dists(r):
        base = (wid * _RPW + r) * _RW
        pltpu.async_copy(dists_hbm.at[pl.ds(base, _RW)], wbuf, semA)

    def wait_a():
        pltpu.make_async_copy(idx_hbm.at[pl.ds(0, _RW)], ibuf, semA).wait()
        pltpu.make_async_copy(dists_hbm.at[pl.ds(0, _RW)], wbuf, semA).wait()

    def issue_b(k, rows, semB):
        for j in range(_W // 128):
            pltpu.async_copy(
                feat_hbm.at[ibuf.at[pl.ds(k * _W + j * 128, 128)]],
                rows.at[pl.ds(j * 128, 128)], semB)

    def wait_b(rows, semB):
        for j in range(_W // 128):
            pltpu.make_async_copy(
                feat_hbm.at[ibuf.at[pl.ds(0, 128)]],
                rows.at[pl.ds(j * 128, 128)], semB).wait()

    def issue_d(r):
        base = (wid * _RPW + r) * _OROW
        pltpu.async_copy(obuf, out_hbm.at[pl.ds(base, _OROW)], semD)

    def wait_d():
        pltpu.make_async_copy(obuf, out_hbm.at[pl.ds(0, _OROW)], semD).wait()

    def weights():
        @plsc.parallel_loop(0, _NG, unroll=1)
        def wgroup(g):
            o = g * 16
            ws = []
            for k in range(_K):
                iv = ibuf[pl.ds(k * _W + o, 16)]
                r = plsc.load_gather(radii_v, [iv])
                ws.append(1.0 - wbuf[pl.ds(k * _W + o, 16)] / (r * r))
            pairs = ws
            while len(pairs) > 1:
                pairs = [a + b for a, b in zip(pairs[0::2], pairs[1::2])]
            rcp = 1.0 / jnp.maximum(pairs[0], 1e-10)
            for k in range(_K):
                wbuf[pl.ds(k * _W + o, 16)] = ws[k] * rcp

    def accum(k, rows):
        first = k == 0

        @plsc.parallel_loop(0, _NG, unroll=1)
        def _wg(g):
            o = g * 16
            wk = wbuf[pl.ds(k * _W + o, 16)]
            rowv = lane + o
            for ch in range(_CW):
                col = plsc.load_gather(rows, [rowv, zvec + ch])
                ev, od = plsc.unpack(plsc.bitcast(col, jnp.bfloat16),
                                     format=plsc.PackFormat.INTERLEAVED)
                pe, po = wk * ev, wk * od
                se = obuf.at[pl.ds(2 * ch * _W + o, 16)]
                so = obuf.at[pl.ds((2 * ch + 1) * _W + o, 16)]
                if first:
                    se[...] = pe
                    so[...] = po
                else:
                    plsc.addupdate(se, pe)
                    plsc.addupdate(so, po)

    issue_a_idx(0)
    issue_a_dists(0)

    def row(r, carry):
        wait_a()
        issue_b(0, rows0, semB0)
        weights()
        issue_b(1, rows1, semB1)

        @pl.when(r > 0)
        def _():
            wait_d()

        for k in range(_K):
            rows, semB = (rows0, semB0) if k % 2 == 0 else (rows1, semB1)
            wait_b(rows, semB)
            accum(k, rows)
            if k + 2 < _K:
                issue_b(k + 2, rows, semB)
            if k == _K - 3:
                @pl.when(r < _RPW - 1)
                def _():
                    issue_a_idx(r + 1)

        @pl.when(r < _RPW - 1)
        def _():
            issue_a_dists(r + 1)

        issue_d(r)
        return carry

    lax.fori_loop(0, _RPW, row, 0)
    wait_d()


@jax.jit
def _render(dt, it, radii, feat_packed):
    mesh = plsc.VectorSubcoreMesh(core_axis_name="c", subcore_axis_name="s")
    f = pl.kernel(
        _body,
        out_type=jax.ShapeDtypeStruct((_N * _C,), jnp.float32),
        mesh=mesh,
        scratch_types=[
            pltpu.VMEM((_P,), jnp.float32),
            pltpu.VMEM((_RW,), jnp.int32),
            pltpu.VMEM((_RW,), jnp.float32),
            pltpu.VMEM((_W, _CW), jnp.int32),
            pltpu.VMEM((_W, _CW), jnp.int32),
            pltpu.VMEM((_OROW,), jnp.float32),
            pltpu.SemaphoreType.DMA,
            pltpu.SemaphoreType.DMA,
            pltpu.SemaphoreType.DMA,
            pltpu.SemaphoreType.DMA,
        ],
        compiler_params=pltpu.CompilerParams(
            needs_layout_passes=False, use_tc_tiling_on_sc=False),
    )
    return f(dt, it, radii, feat_packed)


def kernel(dists, idx, radii, features):
    dt = dists.transpose(0, 1, 3, 2).reshape(-1)
    it = idx.transpose(0, 1, 3, 2).reshape(-1)
    feat_packed = lax.bitcast_convert_type(
        features.astype(jnp.bfloat16).reshape(_P, _CW, 2), jnp.int32)
    out = _render(dt, it, radii, feat_packed)
    return out.reshape(_B, _H, _C, _W).transpose(0, 1, 3, 2)
